# baseline (speedup 1.0000x reference)
import functools

import jax
import jax.numpy as jnp
import numpy as np
from jax import lax
from jax.experimental import pallas as pl
from jax.experimental.pallas import tpu as pltpu
from jax.experimental.pallas import tpu_sc as plsc


def _positional_embedding(num_positions, m):
    pos = jnp.arange(num_positions, dtype=jnp.float32)
    exp = jnp.arange(m, dtype=jnp.float32) / m * jnp.log2(jnp.float32(10000.0))
    denom = jnp.exp2(exp)
    arg = pos[:, None] / denom[None, :]
    even = (jnp.arange(m) % 2) == 0
    return jnp.where(even[None, :], jnp.sin(arg), jnp.cos(arg))


@functools.partial(jax.jit, static_argnames=("ch",))
def _sc_embed(idx3, table, pos, *, ch):
    n_chunks = idx3.shape[0]
    hh = ch // 2
    t, m = table.shape
    c = pos.shape[0]
    info = plsc.get_sparse_core_info()
    nc, ns = info.num_cores, info.num_subcores
    nw = nc * ns
    cpw = n_chunks // nw
    n_groups = (cpw + 2) // 3
    scale = np.sqrt(np.float32(t)).astype(np.float32)
    mesh = plsc.VectorSubcoreMesh(core_axis_name="c", subcore_axis_name="s")

    @functools.partial(
        pl.kernel,
        mesh=mesh,
        out_type=jax.ShapeDtypeStruct((n_chunks, ch, m), jnp.float32),
        scratch_types=[
            pltpu.VMEM((cpw, 2, hh), jnp.int32),
            pltpu.VMEM((c, m), jnp.float32),
            pltpu.VMEM((ch, m), jnp.float32),
            pltpu.VMEM((ch, m), jnp.float32),
            pltpu.VMEM((ch, m), jnp.float32),
            pltpu.VMEM((ch, m), jnp.float32),
            pltpu.VMEM((ch, m), jnp.float32),
            pltpu.VMEM((ch, m), jnp.float32),
            pltpu.SemaphoreType.DMA,
            pltpu.SemaphoreType.DMA,
            pltpu.SemaphoreType.DMA,
            pltpu.SemaphoreType.DMA,
            pltpu.SemaphoreType.DMA,
            pltpu.SemaphoreType.DMA,
            pltpu.SemaphoreType.DMA,
            pltpu.SemaphoreType.DMA,
            pltpu.SemaphoreType.DMA,
        ],
    )
    def k(idx_hbm, table_hbm, pos_hbm, out_hbm, idx_v, pos_v,
          in0, in1, in2, ob0, ob1, ob2,
          ga0, ga1, ga2, gb0, gb1, gb2, s0, s1, s2):
        inb = (in0, in1, in2)
        outb = (ob0, ob1, ob2)
        gsema = (ga0, ga1, ga2)
        gsemb = (gb0, gb1, gb2)
        ssem = (s0, s1, s2)
        wid = lax.axis_index("s") * nc + lax.axis_index("c")
        base = wid * cpw
        pltpu.sync_copy(idx_hbm.at[pl.ds(base, cpw)], idx_v)
        pltpu.sync_copy(pos_hbm, pos_v)

        def start_gather(g, b):
            iv = inb[b]
            pltpu.async_copy(table_hbm.at[idx_v.at[g, 0]],
                             iv.at[pl.ds(0, hh)], gsema[b])
            pltpu.async_copy(table_hbm.at[idx_v.at[g, 1]],
                             iv.at[pl.ds(hh, hh)], gsemb[b])

        def wait_gather(g, b):
            iv = inb[b]
            pltpu.make_async_copy(table_hbm.at[idx_v.at[g, 0]],
                                  iv.at[pl.ds(0, hh)], gsema[b]).wait()
            pltpu.make_async_copy(table_hbm.at[idx_v.at[g, 1]],
                                  iv.at[pl.ds(hh, hh)], gsemb[b]).wait()

        start_gather(0, 0)
        start_gather(1, 1)

        def slot(g, b):
            iv, ov = inb[b], outb[b]
            wait_gather(g, b)

            @pl.when(g + 2 < cpw)
            def _():
                start_gather(g + 2, (b + 2) % 3)

            @pl.when(g >= 3)
            def _():
                pltpu.make_async_copy(ov, out_hbm.at[base + g - 3],
                                      ssem[b]).wait()

            poff = ((base + g) % (c // ch)) * ch

            def j_body(j, carry2):
                pj = poff + j
                sls = [pl.ds(l * 16, 16) for l in range(m // 16)]
                rowv = [iv[j, sl] for sl in sls]
                posv = [pos_v[pj, sl] for sl in sls]
                res = [r * scale + p for r, p in zip(rowv, posv)]
                for sl, x in zip(sls, res):
                    ov[j, sl] = x
                return carry2

            lax.fori_loop(0, ch, j_body, 0, unroll=2)
            pltpu.async_copy(ov, out_hbm.at[base + g], ssem[b])

        def group(p, carry):
            for b in range(3):
                g = p * 3 + b

                @pl.when(g < cpw)
                def _():
                    slot(g, b)
            return carry

        lax.fori_loop(0, n_groups, group, 0, unroll=1)
        for g in (cpw - 3, cpw - 2, cpw - 1):
            pltpu.make_async_copy(outb[g % 3], out_hbm.at[base + g],
                                  ssem[g % 3]).wait()

    return k(idx3, table, pos)


def kernel(input, table):
    b, c = input.shape
    t, m = table.shape
    ch = 100
    pos = _positional_embedding(c, m)
    idx3 = input.astype(jnp.int32).reshape(b * c // ch, 2, ch // 2)
    out = _sc_embed(idx3, table, pos, ch=ch)
    return out.reshape(b, c, m)


# speedup vs baseline: 1.0141x; 1.0141x over previous
import functools

import jax
import jax.numpy as jnp
import numpy as np
from jax import lax
from jax.experimental import pallas as pl
from jax.experimental.pallas import tpu as pltpu
from jax.experimental.pallas import tpu_sc as plsc


def _positional_embedding(num_positions, m):
    pos = jnp.arange(num_positions, dtype=jnp.float32)
    exp = jnp.arange(m, dtype=jnp.float32) / m * jnp.log2(jnp.float32(10000.0))
    denom = jnp.exp2(exp)
    arg = pos[:, None] / denom[None, :]
    even = (jnp.arange(m) % 2) == 0
    return jnp.where(even[None, :], jnp.sin(arg), jnp.cos(arg))


@functools.partial(jax.jit, static_argnames=("ch",))
def _sc_embed(idx2, table, pos, *, ch):
    n_chunks = idx2.shape[0]
    t, m = table.shape
    c = pos.shape[0]
    info = plsc.get_sparse_core_info()
    nc, ns = info.num_cores, info.num_subcores
    nw = nc * ns
    cpw = n_chunks // nw
    n_groups = (cpw + 2) // 3
    scale = np.sqrt(np.float32(t)).astype(np.float32)
    mesh = plsc.VectorSubcoreMesh(core_axis_name="c", subcore_axis_name="s")

    @functools.partial(
        pl.kernel,
        mesh=mesh,
        out_type=jax.ShapeDtypeStruct((n_chunks, ch, m), jnp.float32),
        scratch_types=[
            pltpu.VMEM((cpw, ch), jnp.int32),
            pltpu.VMEM((c, m), jnp.float32),
            pltpu.VMEM((ch, m), jnp.float32),
            pltpu.VMEM((ch, m), jnp.float32),
            pltpu.VMEM((ch, m), jnp.float32),
            pltpu.VMEM((ch, m), jnp.float32),
            pltpu.VMEM((ch, m), jnp.float32),
            pltpu.VMEM((ch, m), jnp.float32),
            pltpu.SemaphoreType.DMA,
            pltpu.SemaphoreType.DMA,
            pltpu.SemaphoreType.DMA,
            pltpu.SemaphoreType.DMA,
            pltpu.SemaphoreType.DMA,
            pltpu.SemaphoreType.DMA,
        ],
    )
    def k(idx_hbm, table_hbm, pos_hbm, out_hbm, idx_v, pos_v,
          in0, in1, in2, ob0, ob1, ob2, g0, g1, g2, s0, s1, s2):
        inb = (in0, in1, in2)
        outb = (ob0, ob1, ob2)
        gsem = (g0, g1, g2)
        ssem = (s0, s1, s2)
        wid = lax.axis_index("s") * nc + lax.axis_index("c")
        base = wid * cpw
        pltpu.sync_copy(idx_hbm.at[pl.ds(base, cpw)], idx_v)
        pltpu.sync_copy(pos_hbm, pos_v)
        pltpu.async_copy(table_hbm.at[idx_v.at[0]], in0, g0)
        pltpu.async_copy(table_hbm.at[idx_v.at[1]], in1, g1)

        def slot(g, b):
            iv, ov = inb[b], outb[b]
            pltpu.make_async_copy(table_hbm.at[idx_v.at[g]], iv, gsem[b]).wait()

            @pl.when(g + 2 < cpw)
            def _():
                pltpu.async_copy(
                    table_hbm.at[idx_v.at[g + 2]], inb[(b + 2) % 3],
                    gsem[(b + 2) % 3])

            @pl.when(g >= 3)
            def _():
                pltpu.make_async_copy(ov, out_hbm.at[base + g - 3],
                                      ssem[b]).wait()

            poff = ((base + g) % (c // ch)) * ch

            def j_body(j, carry2):
                pj = poff + j
                sls = [pl.ds(l * 16, 16) for l in range(m // 16)]
                rowv = [iv[j, sl] for sl in sls]
                posv = [pos_v[pj, sl] for sl in sls]
                res = [r * scale + p for r, p in zip(rowv, posv)]
                for sl, x in zip(sls, res):
                    ov[j, sl] = x
                return carry2

            lax.fori_loop(0, ch, j_body, 0, unroll=2)
            pltpu.async_copy(ov, out_hbm.at[base + g], ssem[b])

        def group(p, carry):
            for b in range(3):
                g = p * 3 + b

                @pl.when(g < cpw)
                def _():
                    slot(g, b)
            return carry

        lax.fori_loop(0, n_groups, group, 0, unroll=1)
        for g in (cpw - 3, cpw - 2, cpw - 1):
            pltpu.make_async_copy(outb[g % 3], out_hbm.at[base + g],
                                  ssem[g % 3]).wait()

    return k(idx2, table, pos)


def kernel(input, table):
    b, c = input.shape
    t, m = table.shape
    ch = 100
    pos = _positional_embedding(c, m)
    idx2 = input.astype(jnp.int32).reshape(b * c // ch, ch)
    out = _sc_embed(idx2, table, pos, ch=ch)
    return out.reshape(b, c, m)
